# Initial kernel scaffold; baseline (speedup 1.0000x reference)
import jax
import jax.numpy as jnp
from jax.experimental import pallas as pl


def kernel(x, edge_index, W_emb, b_emb, W_gcn, b_gcn, W_cls, b_cls):
    raise NotImplementedError("write your pallas kernel here")


# speedup vs baseline: 154.1169x; 154.1169x over previous
import functools

import jax
import jax.numpy as jnp
from jax import lax
from jax.experimental import pallas as pl
from jax.experimental.pallas import tpu as pltpu
from jax.experimental.pallas import tpu_sc as plsc

_E = 320000
_EROWS = _E // 128
_NW = 32
_PROWS = 2560
_ROWS = _PROWS // _NW
_NPAD = 10240


def _sc_histograms(edges3, zeros_n):
  mesh = plsc.VectorSubcoreMesh(core_axis_name="c", subcore_axis_name="s")

  @functools.partial(
      pl.kernel,
      out_type=(
          jax.ShapeDtypeStruct((2, _NPAD), jnp.int32),
          jax.ShapeDtypeStruct((2, _NPAD), jnp.int32),
      ),
      mesh=mesh,
      scratch_types=[
          pltpu.VMEM((_ROWS, 128), jnp.int32),
          pltpu.VMEM((_ROWS, 128), jnp.int32),
          pltpu.VMEM((_ROWS, 128), jnp.int32),
          pltpu.VMEM((_ROWS, 128), jnp.int32),
          pltpu.VMEM_SHARED((_NPAD,), jnp.int32),
          pltpu.VMEM_SHARED((_NPAD,), jnp.int32),
      ],
  )
  def hist_kernel(edges_hbm, zeros_hbm, out_indeg, out_cnt0,
                  src_v, dst_v, ones_v, val_v, hist_d, hist_c):
    c = lax.axis_index("c")
    s = lax.axis_index("s")
    wid = s * 2 + c

    @pl.when(s == 0)
    def _():
      pltpu.sync_copy(zeros_hbm, hist_d)
      pltpu.sync_copy(zeros_hbm, hist_c)

    base = wid * _ROWS
    pltpu.sync_copy(edges_hbm.at[0, pl.ds(base, _ROWS)], src_v)
    pltpu.sync_copy(edges_hbm.at[1, pl.ds(base, _ROWS)], dst_v)

    one16 = jnp.full((16,), 1, jnp.int32)

    def row_body(r, carry):
      for j in range(8):
        sl = pl.ds(j * 16, 16)
        d16 = dst_v[r, sl]
        ones_v[r, sl] = one16
        val_v[r, sl] = jnp.where(d16 == 0, 1, 0).astype(jnp.int32)
      return carry

    lax.fori_loop(0, _ROWS, row_body, 0)

    plsc.subcore_barrier()

    def scatter_body(r, carry):
      pltpu.sync_copy(ones_v.at[r], hist_d.at[dst_v.at[r]], add=True)
      pltpu.sync_copy(val_v.at[r], hist_c.at[src_v.at[r]], add=True)
      return carry

    lax.fori_loop(0, _ROWS, scatter_body, 0)

    plsc.subcore_barrier()

    @pl.when(s == 0)
    def _():
      pltpu.sync_copy(hist_d, out_indeg.at[c])
      pltpu.sync_copy(hist_c, out_cnt0.at[c])

  return hist_kernel(edges3, zeros_n)


def _tc_dense(ind_t, cnt_t, x, w_emb, b_emb2, w_gcn, b_gcn2, w_cls, b_cls2):
  nb = x.shape[0]
  nchunk = ind_t.shape[2]

  def body(ind_ref, cnt_ref, x_ref, we_ref, be_ref, wg_ref, bg_ref,
           wc_ref, bc_ref, o_ref):
    ind = (ind_ref[0] + ind_ref[1]).astype(jnp.float32)
    cnt = (cnt_ref[0] + cnt_ref[1]).astype(jnp.float32)
    xx = x_ref[:]
    e0 = jnp.maximum(
        jnp.dot(xx, we_ref[:], preferred_element_type=jnp.float32)
        + be_ref[:], 0.0)
    h1 = jnp.dot(e0, wg_ref[:], preferred_element_type=jnp.float32)
    bg = bg_ref[:]

    s_rows = []
    for b in range(nb):
      hb = h1[b:b + 1, :]
      acc = jnp.zeros((1, 128), jnp.float32)
      for i in range(nchunk):
        col = lax.slice(ind, (0, i), (128, i + 1))
        wcol = lax.slice(cnt, (0, i), (128, i + 1))
        z = jnp.maximum(col * hb + bg, 0.0)
        acc = acc + jnp.sum(wcol * z, axis=0, keepdims=True)
      s_rows.append(acc)
    sm = jnp.concatenate(s_rows, axis=0)
    out2 = jnp.maximum(
        jnp.dot(sm, wg_ref[:], preferred_element_type=jnp.float32) + bg, 0.0)
    y = jnp.dot(out2, wc_ref[:], preferred_element_type=jnp.float32) + bc_ref[:]
    o_ref[:] = y

  return pl.pallas_call(
      body,
      out_shape=jax.ShapeDtypeStruct((nb, 1), jnp.float32),
  )(ind_t, cnt_t, x, w_emb, b_emb2, w_gcn, b_gcn2, w_cls, b_cls2)


def kernel(x, edge_index, W_emb, b_emb, W_gcn, b_gcn, W_cls, b_cls):
  ei = edge_index.astype(jnp.int32)
  npadrows = _PROWS - _EROWS
  pad_dst = jnp.broadcast_to(
      jnp.arange(10112, 10240, dtype=jnp.int32)[None, :], (npadrows, 128))
  pad_src = jnp.zeros((npadrows, 128), jnp.int32)
  pad = jnp.stack([pad_src, pad_dst])
  edges3 = jnp.concatenate([ei.reshape(2, _EROWS, 128), pad], axis=1)
  zeros_n = jnp.zeros((_NPAD,), jnp.int32)
  ind2, cnt2 = _sc_histograms(edges3, zeros_n)
  ind_t = ind2.reshape(2, _NPAD // 128, 128).transpose(0, 2, 1)
  cnt_t = cnt2.reshape(2, _NPAD // 128, 128).transpose(0, 2, 1)
  d = x.shape[1]
  return _tc_dense(
      ind_t, cnt_t, x, W_emb, b_emb.reshape(1, d), W_gcn,
      b_gcn.reshape(1, d), W_cls, b_cls.reshape(1, 1))


# speedup vs baseline: 160.3743x; 1.0406x over previous
import functools

import jax
import jax.numpy as jnp
from jax import lax
from jax.experimental import pallas as pl
from jax.experimental.pallas import tpu as pltpu
from jax.experimental.pallas import tpu_sc as plsc

_E = 320000
_NW = 32
_CH = _E // _NW
_NPAD = 10240
_ZCH = _NPAD // 16


def _sc_histograms(src_flat, dst_flat):
  mesh = plsc.VectorSubcoreMesh(core_axis_name="c", subcore_axis_name="s")

  @functools.partial(
      pl.kernel,
      out_type=(
          jax.ShapeDtypeStruct((2, _NPAD), jnp.int32),
          jax.ShapeDtypeStruct((2, _NPAD), jnp.int32),
      ),
      mesh=mesh,
      scratch_types=[
          pltpu.VMEM((_CH,), jnp.int32),
          pltpu.VMEM((_CH,), jnp.int32),
          pltpu.VMEM((_CH,), jnp.int32),
          pltpu.VMEM((_ZCH,), jnp.int32),
          pltpu.VMEM((_CH,), jnp.int32),
          pltpu.VMEM_SHARED((_NPAD,), jnp.int32),
          pltpu.VMEM_SHARED((_NPAD,), jnp.int32),
          pltpu.SemaphoreType.DMA,
          pltpu.SemaphoreType.DMA,
      ],
  )
  def hist_kernel(src_hbm, dst_hbm, out_indeg, out_cnt0,
                  src_v, dst_v, ones_v, zero_v, val_v,
                  hist_d, hist_c, sem_a, sem_b):
    c = lax.axis_index("c")
    s = lax.axis_index("s")
    wid = s * 2 + c
    base = wid * _CH

    cp_src = pltpu.async_copy(src_hbm.at[pl.ds(base, _CH)], src_v, sem_a)
    cp_dst = pltpu.async_copy(dst_hbm.at[pl.ds(base, _CH)], dst_v, sem_b)

    zero16 = jnp.full((16,), 0, jnp.int32)
    one16 = jnp.full((16,), 1, jnp.int32)

    def zero_body(i, carry):
      zero_v[pl.ds(i * 16, 16)] = zero16
      return carry

    lax.fori_loop(0, _ZCH // 16, zero_body, 0)
    pltpu.sync_copy(zero_v, hist_d.at[pl.ds(s * _ZCH, _ZCH)])
    pltpu.sync_copy(zero_v, hist_c.at[pl.ds(s * _ZCH, _ZCH)])

    def ones_body(i, carry):
      ones_v[pl.ds(i * 16, 16)] = one16
      return carry

    lax.fori_loop(0, _CH // 16, ones_body, 0)

    cp_dst.wait()

    def val_body(i, carry):
      sl = pl.ds(i * 16, 16)
      val_v[sl] = jnp.where(dst_v[sl] == 0, 1, 0).astype(jnp.int32)
      return carry

    lax.fori_loop(0, _CH // 16, val_body, 0)

    cp_src.wait()
    plsc.subcore_barrier()

    sc_d = pltpu.async_copy(ones_v, hist_d.at[dst_v], sem_b, add=True)
    sc_c = pltpu.async_copy(val_v, hist_c.at[src_v], sem_a, add=True)
    sc_d.wait()
    sc_c.wait()
    plsc.subcore_barrier()

    @pl.when(s == 0)
    def _():
      pltpu.sync_copy(hist_d, out_indeg.at[c])
      pltpu.sync_copy(hist_c, out_cnt0.at[c])

  return hist_kernel(src_flat, dst_flat)


def _tc_dense(ind3, cnt3, x, w_emb, b_emb2, w_gcn, b_gcn2, w_cls, b_cls2):
  nb = x.shape[0]
  nchunk = ind3.shape[1]

  def body(ind_ref, cnt_ref, x_ref, we_ref, be_ref, wg_ref, bg_ref,
           wc_ref, bc_ref, o_ref):
    ind = jnp.transpose(
        (ind_ref[0] + ind_ref[1]).astype(jnp.float32))
    cnt = jnp.transpose(
        (cnt_ref[0] + cnt_ref[1]).astype(jnp.float32))
    xx = x_ref[:]
    e0 = jnp.maximum(
        jnp.dot(xx, we_ref[:], preferred_element_type=jnp.float32)
        + be_ref[:], 0.0)
    h1 = jnp.dot(e0, wg_ref[:], preferred_element_type=jnp.float32)
    bg = bg_ref[:]

    s_rows = []
    for b in range(nb):
      hb = h1[b:b + 1, :]
      acc = jnp.zeros((1, 128), jnp.float32)
      for i in range(nchunk):
        col = lax.slice(ind, (0, i), (128, i + 1))
        wcol = lax.slice(cnt, (0, i), (128, i + 1))
        z = jnp.maximum(col * hb + bg, 0.0)
        acc = acc + jnp.sum(wcol * z, axis=0, keepdims=True)
      s_rows.append(acc)
    sm = jnp.concatenate(s_rows, axis=0)
    out2 = jnp.maximum(
        jnp.dot(sm, wg_ref[:], preferred_element_type=jnp.float32) + bg, 0.0)
    y = jnp.dot(out2, wc_ref[:], preferred_element_type=jnp.float32) + bc_ref[:]
    o_ref[:] = y

  return pl.pallas_call(
      body,
      out_shape=jax.ShapeDtypeStruct((nb, 1), jnp.float32),
  )(ind3, cnt3, x, w_emb, b_emb2, w_gcn, b_gcn2, w_cls, b_cls2)


def kernel(x, edge_index, W_emb, b_emb, W_gcn, b_gcn, W_cls, b_cls):
  ei = edge_index.astype(jnp.int32)
  ind2, cnt2 = _sc_histograms(ei[0], ei[1])
  ind3 = ind2.reshape(2, _NPAD // 128, 128)
  cnt3 = cnt2.reshape(2, _NPAD // 128, 128)
  d = x.shape[1]
  return _tc_dense(
      ind3, cnt3, x, W_emb, b_emb.reshape(1, d), W_gcn,
      b_gcn.reshape(1, d), W_cls, b_cls.reshape(1, 1))


# speedup vs baseline: 185.5137x; 1.1568x over previous
import functools

import jax
import jax.numpy as jnp
from jax import lax
from jax.experimental import pallas as pl
from jax.experimental.pallas import tpu as pltpu
from jax.experimental.pallas import tpu_sc as plsc

_E = 320000
_NW = 32
_MAIN = 9984
_REM = _E - _MAIN * _NW
_FLAT = _MAIN + _REM
_NPAD = 10240
_ZCH = _NPAD // 16
_DUMP = _NPAD - 2


def _sc_histograms(ei):
  mesh = plsc.VectorSubcoreMesh(core_axis_name="c", subcore_axis_name="s")

  @functools.partial(
      pl.kernel,
      out_type=(
          jax.ShapeDtypeStruct((2, _NPAD), jnp.int32),
          jax.ShapeDtypeStruct((2, _NPAD), jnp.int32),
      ),
      mesh=mesh,
      scratch_types=[
          pltpu.VMEM((_FLAT,), jnp.int32),
          pltpu.VMEM((_FLAT,), jnp.int32),
          pltpu.VMEM((_FLAT,), jnp.int32),
          pltpu.VMEM((_FLAT,), jnp.int32),
          pltpu.VMEM((_ZCH,), jnp.int32),
          pltpu.VMEM_SHARED((_NPAD,), jnp.int32),
          pltpu.VMEM_SHARED((_NPAD,), jnp.int32),
          pltpu.SemaphoreType.DMA,
          pltpu.SemaphoreType.DMA,
      ],
  )
  def hist_kernel(ei_hbm, out_indeg, out_cnt0,
                  src_v, dst_v, ones_v, val_v, zero_v,
                  hist_d, hist_c, sem_a, sem_b):
    c = lax.axis_index("c")
    s = lax.axis_index("s")
    wid = s * 2 + c
    base = wid * _MAIN

    cp_src = pltpu.async_copy(
        ei_hbm.at[0, pl.ds(base, _MAIN)], src_v.at[pl.ds(0, _MAIN)], sem_a)
    cp_dst = pltpu.async_copy(
        ei_hbm.at[1, pl.ds(base, _MAIN)], dst_v.at[pl.ds(0, _MAIN)], sem_b)

    @pl.when(wid == _NW - 1)
    def _():
      pltpu.sync_copy(ei_hbm.at[0, pl.ds(_MAIN * _NW, _REM)],
                      src_v.at[pl.ds(_MAIN, _REM)])
      pltpu.sync_copy(ei_hbm.at[1, pl.ds(_MAIN * _NW, _REM)],
                      dst_v.at[pl.ds(_MAIN, _REM)])

    zero16 = jnp.full((16,), 0, jnp.int32)
    one16 = jnp.full((16,), 1, jnp.int32)

    @pl.when(wid != _NW - 1)
    def _():
      dump16 = jnp.full((16,), _DUMP, jnp.int32)

      def pad_body(i, carry):
        sl = pl.ds(_MAIN + i * 16, 16)
        src_v[sl] = zero16
        dst_v[sl] = dump16
        return carry

      lax.fori_loop(0, _REM // 16, pad_body, 0)

    def zero_body(i, carry):
      zero_v[pl.ds(i * 16, 16)] = zero16
      return carry

    lax.fori_loop(0, _ZCH // 16, zero_body, 0)
    pltpu.sync_copy(zero_v, hist_d.at[pl.ds(s * _ZCH, _ZCH)])
    pltpu.sync_copy(zero_v, hist_c.at[pl.ds(s * _ZCH, _ZCH)])

    def ones_body(i, carry):
      ones_v[pl.ds(i * 16, 16)] = one16
      return carry

    lax.fori_loop(0, _FLAT // 16, ones_body, 0)

    cp_dst.wait()
    plsc.subcore_barrier()

    sc_d = pltpu.async_copy(ones_v, hist_d.at[dst_v], sem_b, add=True)

    def val_body(i, carry):
      sl = pl.ds(i * 16, 16)
      val_v[sl] = jnp.where(dst_v[sl] == 0, 1, 0).astype(jnp.int32)
      return carry

    lax.fori_loop(0, _FLAT // 16, val_body, 0)

    cp_src.wait()
    sc_c = pltpu.async_copy(val_v, hist_c.at[src_v], sem_a, add=True)
    sc_d.wait()
    sc_c.wait()
    plsc.subcore_barrier()

    @pl.when(s == 0)
    def _():
      pltpu.sync_copy(hist_d, out_indeg.at[c])
      pltpu.sync_copy(hist_c, out_cnt0.at[c])

  return hist_kernel(ei)


def _tc_dense(ind2, cnt2, x, w_emb, b_emb2, w_gcn, b_gcn2, w_cls, b_cls2):
  nb = x.shape[0]
  d = x.shape[1]

  def body(ind_ref, cnt_ref, x_ref, we_ref, be_ref, wg_ref, bg_ref,
           wc_ref, bc_ref, o_ref):
    xx = x_ref[:]
    e0 = jnp.maximum(
        jnp.dot(xx, we_ref[:], preferred_element_type=jnp.float32)
        + be_ref[:], 0.0)
    h1 = jnp.dot(e0, wg_ref[:], preferred_element_type=jnp.float32)
    bg = bg_ref[:]
    h2 = jnp.concatenate([h1[b:b + 1, :] for b in range(nb)], axis=1)
    bg2 = jnp.concatenate([bg] * nb, axis=1)
    h2_hi = h2.astype(jnp.bfloat16).astype(jnp.float32)
    h2_lo = h2 - h2_hi

    ind_full = (ind_ref[0:1, :] + ind_ref[1:2, :]).astype(jnp.float32)
    cnt_full = (cnt_ref[0:1, :] + cnt_ref[1:2, :]).astype(jnp.float32)
    dn = (((0,), (0,)), ((), ()))
    outer = (
        lax.dot_general(ind_full, h2_hi, dn,
                        preferred_element_type=jnp.float32)
        + lax.dot_general(ind_full, h2_lo, dn,
                          preferred_element_type=jnp.float32))
    z = jnp.maximum(outer + bg2, 0.0)
    acc = jnp.dot(cnt_full, z, preferred_element_type=jnp.float32)

    sm = jnp.concatenate(
        [acc[:, b * 128:(b + 1) * 128] for b in range(nb)], axis=0)
    out2 = jnp.maximum(
        jnp.dot(sm, wg_ref[:], preferred_element_type=jnp.float32) + bg, 0.0)
    y = jnp.dot(out2, wc_ref[:], preferred_element_type=jnp.float32) + bc_ref[:]
    o_ref[:] = y

  return pl.pallas_call(
      body,
      out_shape=jax.ShapeDtypeStruct((nb, 1), jnp.float32),
  )(ind2, cnt2, x, w_emb, b_emb2, w_gcn, b_gcn2, w_cls, b_cls2)


def kernel(x, edge_index, W_emb, b_emb, W_gcn, b_gcn, W_cls, b_cls):
  ei = edge_index.astype(jnp.int32)
  ind2, cnt2 = _sc_histograms(ei)
  d = x.shape[1]
  return _tc_dense(
      ind2, cnt2, x, W_emb, b_emb.reshape(1, d), W_gcn,
      b_gcn.reshape(1, d), W_cls, b_cls.reshape(1, 1))
